# baseline (speedup 1.0000x reference)
import functools

import jax
import jax.numpy as jnp
from jax.experimental import pallas as pl
from jax.experimental.pallas import tpu as pltpu

K_BLK = 2048
LANES = 128


def _nn_kernel(x_ref, db_ref, dist_ref, idx_ref,
               mv128, mi128, *, k_total, nblk):
    blk = pl.program_id(0)
    nchunk = K_BLK // LANES

    @pl.when(blk == 0)
    def _init():
        mv128[...] = jnp.full_like(mv128, jnp.inf)
        mi128[...] = jnp.zeros_like(mi128)

    tail_last = k_total - (k_total // K_BLK) * K_BLK
    if tail_last:
        @pl.when(blk == nblk - 1)
        def _zero_tail():
            db_ref[tail_last:, :] = jnp.zeros(
                (K_BLK - tail_last, db_ref.shape[1]), jnp.float32)

    xb = x_ref[...]
    dbb = db_ref[...]

    s = jax.lax.dot_general(
        xb, dbb, (((1,), (1,)), ((), ())),
        preferred_element_type=jnp.float32)

    ones = jnp.ones((1, xb.shape[1]), jnp.float32)
    d2 = jax.lax.dot_general(
        ones, dbb * dbb, (((1,), (1,)), ((), ())),
        precision=jax.lax.Precision.HIGHEST,
        preferred_element_type=jnp.float32)
    tail = k_total - blk * K_BLK
    iota_row = jax.lax.broadcasted_iota(jnp.int32, d2.shape, 1)
    d2 = jnp.where(iota_row < tail, d2, jnp.inf)

    x2 = jnp.sum(xb * xb, axis=1, keepdims=True)

    run_v = (x2 + d2[:, 0:LANES]) - 2.0 * s[:, 0:LANES]
    run_i = jnp.full(run_v.shape, 0, jnp.int32)
    for c in range(1, nchunk):
        v = (x2 + d2[:, c * LANES:(c + 1) * LANES]) \
            - 2.0 * s[:, c * LANES:(c + 1) * LANES]
        m = v < run_v
        run_v = jnp.where(m, v, run_v)
        run_i = jnp.where(m, c, run_i)
    lane = jax.lax.broadcasted_iota(jnp.int32, run_v.shape, 1)
    gidx = (blk * K_BLK + lane) + run_i * LANES

    m = run_v < mv128[...]
    mi128[...] = jnp.where(m, gidx, mi128[...])
    mv128[...] = jnp.where(m, run_v, mv128[...])

    @pl.when(blk == nblk - 1)
    def _finish():
        mv = mv128[...]
        gmin = jnp.min(mv, axis=1, keepdims=True)
        cand = jnp.where(mv == gmin, mi128[...], k_total)
        dist_ref[...] = jnp.sqrt(jnp.maximum(gmin, 0.0))
        idx_ref[...] = jnp.min(cand, axis=1, keepdims=True)


def kernel(x, db):
    q, d = x.shape
    k_total = db.shape[0]
    nblk = pl.cdiv(k_total, K_BLK)

    out_dist, out_idx = pl.pallas_call(
        functools.partial(_nn_kernel, k_total=k_total, nblk=nblk),
        grid=(nblk,),
        in_specs=[
            pl.BlockSpec((q, d), lambda i: (0, 0)),
            pl.BlockSpec((K_BLK, d), lambda i: (i, 0)),
        ],
        out_specs=[
            pl.BlockSpec((q, 1), lambda i: (0, 0)),
            pl.BlockSpec((q, 1), lambda i: (0, 0)),
        ],
        out_shape=[
            jax.ShapeDtypeStruct((q, 1), jnp.float32),
            jax.ShapeDtypeStruct((q, 1), jnp.int32),
        ],
        scratch_shapes=[
            pltpu.VMEM((q, LANES), jnp.float32),
            pltpu.VMEM((q, LANES), jnp.int32),
        ],
        compiler_params=pltpu.CompilerParams(
            dimension_semantics=("arbitrary",)),
    )(x, db)

    return (out_dist.reshape(q), out_idx.reshape(q))


# speedup vs baseline: 1.6242x; 1.6242x over previous
import functools

import jax
import jax.numpy as jnp
from jax.experimental import pallas as pl
from jax.experimental.pallas import tpu as pltpu

K_BLK = 4096


def _nn_kernel(x_ref, db_ref, dist_ref, idx_ref, minval, minidx, *, k_total):
    blk = pl.program_id(0)
    nblk = pl.num_programs(0)

    @pl.when(blk == 0)
    def _init():
        minval[...] = jnp.full_like(minval, jnp.inf)
        minidx[...] = jnp.zeros_like(minidx)

    tail_last = k_total - (k_total // K_BLK) * K_BLK
    if tail_last:
        @pl.when(blk == nblk - 1)
        def _zero_tail():
            db_ref[tail_last:, :] = jnp.zeros(
                (K_BLK - tail_last, db_ref.shape[1]), jnp.float32)

    xb = x_ref[...]
    dbb = db_ref[...]

    s = jax.lax.dot_general(
        xb, dbb, (((1,), (1,)), ((), ())),
        preferred_element_type=jnp.float32)

    ones = jnp.ones((1, xb.shape[1]), jnp.float32)
    d2 = jax.lax.dot_general(
        ones, dbb * dbb, (((1,), (1,)), ((), ())),
        preferred_element_type=jnp.float32)
    x2 = jnp.sum(xb * xb, axis=1, keepdims=True)

    tail = k_total - blk * K_BLK
    iota_row = jax.lax.broadcasted_iota(jnp.int32, d2.shape, 1)
    d2 = jnp.where(iota_row < tail, d2, jnp.inf)

    dist2 = (x2 + d2) - 2.0 * s

    bmin = jnp.min(dist2, axis=1, keepdims=True)
    barg = jnp.argmin(dist2, axis=1, keepdims=True).astype(jnp.int32) \
        + blk * K_BLK

    better = bmin < minval[...]
    minidx[...] = jnp.where(better, barg, minidx[...])
    minval[...] = jnp.where(better, bmin, minval[...])

    @pl.when(blk == nblk - 1)
    def _finish():
        dist_ref[...] = jnp.sqrt(jnp.maximum(minval[...], 0.0))
        idx_ref[...] = minidx[...]


def kernel(x, db):
    q, d = x.shape
    k_total = db.shape[0]
    nblk = pl.cdiv(k_total, K_BLK)

    out_dist, out_idx = pl.pallas_call(
        functools.partial(_nn_kernel, k_total=k_total),
        grid=(nblk,),
        in_specs=[
            pl.BlockSpec((q, d), lambda i: (0, 0)),
            pl.BlockSpec((K_BLK, d), lambda i: (i, 0)),
        ],
        out_specs=[
            pl.BlockSpec((q, 1), lambda i: (0, 0)),
            pl.BlockSpec((q, 1), lambda i: (0, 0)),
        ],
        out_shape=[
            jax.ShapeDtypeStruct((q, 1), jnp.float32),
            jax.ShapeDtypeStruct((q, 1), jnp.int32),
        ],
        scratch_shapes=[
            pltpu.VMEM((q, 1), jnp.float32),
            pltpu.VMEM((q, 1), jnp.int32),
        ],
        compiler_params=pltpu.CompilerParams(
            dimension_semantics=("arbitrary",)),
    )(x, db)

    return (out_dist.reshape(q), out_idx.reshape(q))
